# baseline (speedup 1.0000x reference)
import functools

import jax
import jax.numpy as jnp
from jax import lax
from jax.experimental import pallas as pl
from jax.experimental.pallas import tpu as pltpu
from jax.experimental.pallas import tpu_sc as plsc

D_MODEL = 768
E = 64
CHUNK = 2048
_SIGN = -2147483648


def _key_of(a):
    ki = lax.bitcast_convert_type(a, jnp.int32)
    ku = lax.bitcast_convert_type(a, jnp.uint32)
    kun = jnp.where(ki < 0, ~ku, ku | jnp.uint32(0x80000000))
    return lax.bitcast_convert_type(kun, jnp.int32) ^ jnp.int32(_SIGN)


def _unkey(k):
    ku = lax.bitcast_convert_type(k ^ jnp.int32(_SIGN), jnp.uint32)
    top = (ku & jnp.uint32(0x80000000)) != 0
    u = jnp.where(top, ku ^ jnp.uint32(0x80000000), ~ku)
    return lax.bitcast_convert_type(u, jnp.float32)


def _unkey_sc(k):
    pat = k ^ jnp.int32(_SIGN)
    fb = jnp.where(k >= 0, pat & jnp.int32(0x7FFFFFFF), ~pat)
    return lax.bitcast_convert_type(fb, jnp.float32)


def _matmul_body(x0_ref, x1_ref, w_ref, keyf_ref, keye_ref):
    dn = (((1,), (1,)), ((), ()))
    a0 = lax.dot_general(x0_ref[...], w_ref[...], dimension_numbers=dn,
                         preferred_element_type=jnp.float32)
    a1 = lax.dot_general(x1_ref[...], w_ref[...], dimension_numbers=dn,
                         preferred_element_type=jnp.float32)
    k0 = _key_of(a0)
    k1 = _key_of(a1)
    keyf_ref[...] = jnp.concatenate([k0, k1], axis=1)
    keye_ref[0] = k0.T
    keye_ref[1] = k1.T


def _affinity_key(tokens, w_sel, *, interpret=False):
    n, d = tokens.shape
    half = n // 2
    T = min(2048, half)
    nb = half // T
    return pl.pallas_call(
        _matmul_body,
        grid=(nb,),
        in_specs=[
            pl.BlockSpec((T, d), lambda i: (i, 0)),
            pl.BlockSpec((T, d), lambda i: (i + nb, 0)),
            pl.BlockSpec((E, d), lambda i: (0, 0)),
        ],
        out_specs=[
            pl.BlockSpec((T, 2 * E), lambda i: (i, 0)),
            pl.BlockSpec((2, E, T), lambda i: (0, 0, i)),
        ],
        out_shape=[
            jax.ShapeDtypeStruct((half, 2 * E), jnp.int32),
            jax.ShapeDtypeStruct((2, E, half), jnp.int32),
        ],
        interpret=interpret,
    )(tokens, tokens, w_sel)


def _splat_sum(v):
    return plsc.cummax(lax.rev(plsc.cumsum(v), (0,)))


def _splat_max(v):
    return plsc.cummax(lax.rev(plsc.cummax(v), (0,)))


def _sc_select(keye, cap):
    _, e, half = keye.shape
    n = 2 * half
    nv = n // 16
    U = 8
    mesh = plsc.VectorSubcoreMesh(core_axis_name="c", subcore_axis_name="s")

    @functools.partial(
        pl.kernel,
        out_type=[
            jax.ShapeDtypeStruct((32, 16), jnp.int32),
            jax.ShapeDtypeStruct((32, 16), jnp.float32),
        ],
        scratch_types=[
            pltpu.VMEM((n,), jnp.int32),
            pltpu.VMEM((n,), jnp.int32),
            pltpu.VMEM((16, 256), jnp.int32),
            pltpu.VMEM((256,), jnp.int32),
            pltpu.VMEM((16,), jnp.int32),
            pltpu.VMEM((16,), jnp.float32),
            pltpu.SemaphoreType.DMA,
            pltpu.SemaphoreType.DMA,
        ],
        compiler_params=pltpu.CompilerParams(needs_layout_passes=False),
        mesh=mesh,
    )
    def body(keye_hbm, out_i, out_f, row0, row1, hist, tot, rowi_v, rowf_v,
             sem0, sem1):
        wid = lax.axis_index("s") * 2 + lax.axis_index("c")
        e0 = 2 * wid
        lanes = lax.iota(jnp.int32, 16)
        ones = jnp.ones((16,), jnp.int32)

        c00 = pltpu.async_copy(keye_hbm.at[0, e0], row0.at[pl.ds(0, half)], sem0)
        c01 = pltpu.async_copy(keye_hbm.at[1, e0], row0.at[pl.ds(half, half)], sem0)
        c10 = pltpu.async_copy(keye_hbm.at[0, e0 + 1], row1.at[pl.ds(0, half)], sem1)
        c11 = pltpu.async_copy(keye_hbm.at[1, e0 + 1], row1.at[pl.ds(half, half)], sem1)

        def zero_hist():
            def zcol(i, carry):
                for l in range(16):
                    hist[l, pl.ds(i * 16, 16)] = jnp.zeros((16,), jnp.int32)
                return carry
            lax.fori_loop(0, 16, zcol, 0)

        def digit_find(needv):
            scs = jnp.zeros((16,), jnp.int32)

            def totals(i, scs):
                acc = hist[0, pl.ds(i * 16, 16)]
                for l in range(1, 16):
                    acc = acc + hist[l, pl.ds(i * 16, 16)]
                tot[pl.ds(i * 16, 16)] = acc
                return scs + acc

            scs = lax.fori_loop(0, 16, totals, scs)
            rv = lax.rev(scs, (0,))
            ic = plsc.cumsum(rv)
            ex = ic - rv
            cond = jnp.logical_and(ex < needv, ic >= needv)
            c_star = _splat_sum(jnp.where(cond, 15 - lanes, 0))
            ssuf = _splat_sum(jnp.where(cond, ex, 0))
            t16 = plsc.load_gather(tot, [lanes * 16 + c_star])
            tr = lax.rev(t16, (0,))
            tc = plsc.cumsum(tr)
            te = tc - tr
            before = ssuf + te
            cond2 = jnp.logical_and(before < needv, before + tr >= needv)
            digit = _splat_sum(jnp.where(cond2, c_star * 16 + 15 - lanes, 0))
            above = _splat_sum(jnp.where(cond2, before, 0))
            return digit, above

        def swap_digit(d):
            return ((d & jnp.int32(15)) << 4) | lax.shift_right_logical(d, 4)

        def process(row):
            zero_hist()

            def scan0(i, m):
                for u in range(U):
                    k = row[pl.ds(i * (16 * U) + u * 16, 16)]
                    pat = k ^ jnp.int32(_SIGN)
                    d = lax.shift_right_logical(pat, 24)
                    plsc.addupdate_scatter(hist, [lanes, swap_digit(d)], ones)
                    m = jnp.maximum(m, k)
                return m

            m = lax.fori_loop(0, nv // U, scan0,
                              jnp.full((16,), _SIGN, jnp.int32))
            mxk = _splat_max(m)

            needv = jnp.full((16,), cap, jnp.int32)
            digit, above = digit_find(needv)
            pfx = digit
            needv = needv - above

            for p in range(1, 4):
                shift = 24 - 8 * p
                zero_hist()

                def scanp(i, carry, shift=shift, pfx=pfx, row=row):
                    for u in range(U):
                        k = row[pl.ds(i * (16 * U) + u * 16, 16)]
                        pat = k ^ jnp.int32(_SIGN)
                        d = lax.shift_right_logical(pat, shift) & jnp.int32(255)
                        msk = lax.shift_right_logical(pat, shift + 8) == pfx
                        plsc.addupdate_scatter(hist, [lanes, swap_digit(d)],
                                               ones, mask=msk)
                    return carry

                lax.fori_loop(0, nv // U, scanp, 0)
                digit, above = digit_find(needv)
                pfx = (pfx << 8) | digit
                needv = needv - above

            thr_s = pfx ^ jnp.int32(_SIGN)

            mxf = _unkey_sc(mxk)

            def scane(i, acc):
                for u in range(U):
                    k = row[pl.ds(i * (16 * U) + u * 16, 16)]
                    sel = k >= thr_s
                    a = _unkey_sc(k)
                    acc = acc + jnp.where(sel, jnp.exp(a - mxf), 0.0)
                return acc

            acc = lax.fori_loop(0, nv // U, scane,
                                jnp.zeros((16,), jnp.float32))
            return thr_s, mxk, _splat_sum(acc)

        c00.wait()
        c01.wait()
        thr0, mxk0, s0 = process(row0)
        c10.wait()
        c11.wait()
        thr1, mxk1, s1 = process(row1)

        rowi_v[...] = jnp.where(
            lanes == 0, thr0,
            jnp.where(lanes == 1, thr1,
                      jnp.where(lanes == 2, mxk0,
                                jnp.where(lanes == 3, mxk1, 0))))
        rowf_v[...] = jnp.where(lanes == 0, s0,
                                jnp.where(lanes == 1, s1, 0.0))
        pltpu.sync_copy(rowi_v, out_i.at[wid])
        pltpu.sync_copy(rowf_v, out_f.at[wid])

    return body(keye)


def _finalize_body(key_ref, thr_ref, mxk_ref, s_ref, w_out_ref, s_out_ref):
    n2 = key_ref.shape[0]
    chunk = min(CHUNK, n2)
    nchunks = n2 // chunk
    L = 2 * E

    thr = jnp.concatenate([thr_ref[...], thr_ref[...]], axis=1)
    mx = _unkey(jnp.concatenate([mxk_ref[...], mxk_ref[...]], axis=1))
    s = jnp.concatenate([s_ref[...], s_ref[...]], axis=1)

    lane = lax.broadcasted_iota(jnp.int32, (1, L), 1)
    half0 = lane < E

    def out_step(c, carry):
        rows = pl.ds(c * chunk, chunk)
        k = key_ref[rows, :]
        sel = k >= thr
        self_f = sel.astype(jnp.float32)
        p = jnp.where(sel, jnp.exp(_unkey(k) - mx), 0.0)
        cnt0 = jnp.sum(jnp.where(half0, self_f, 0.0), axis=1, keepdims=True)
        cnt1 = jnp.sum(jnp.where(half0, 0.0, self_f), axis=1, keepdims=True)
        div = jnp.maximum(jnp.where(half0, cnt0, cnt1), 1.0)
        w = p / (s * div)
        r0 = pl.ds(c * chunk, chunk)
        r1 = pl.ds(n2 + c * chunk, chunk)
        w_out_ref[r0, :] = w[:, :E]
        w_out_ref[r1, :] = w[:, E:]
        s_out_ref[r0, :] = self_f[:, :E]
        s_out_ref[r1, :] = self_f[:, E:]
        return carry

    lax.fori_loop(0, nchunks, out_step, 0)


def _finalize(key_folded, thr, mxk, s, *, interpret=False):
    n2, l = key_folded.shape
    n = n2 * 2
    return pl.pallas_call(
        _finalize_body,
        out_shape=(
            jax.ShapeDtypeStruct((n, E), jnp.float32),
            jax.ShapeDtypeStruct((n, E), jnp.float32),
        ),
        interpret=interpret,
    )(key_folded, thr, mxk, s)


def kernel(hidden_states, W_sel):
    batch, seq, d = hidden_states.shape
    num_tokens = batch * seq
    cap = int(num_tokens / E)
    c = min(cap, num_tokens)
    tokens = hidden_states.reshape(num_tokens, d)
    keyf, keye = _affinity_key(tokens, W_sel)
    out_i, out_f = _sc_select(keye, c)
    thr = out_i[:, 0:2].reshape(1, E)
    mxk = out_i[:, 2:4].reshape(1, E)
    s = out_f[:, 0:2].reshape(1, E)
    weights, assign = _finalize(keyf, thr, mxk, s)
    return (weights, assign, cap)


# speedup vs baseline: 1.0805x; 1.0805x over previous
import functools

import jax
import jax.numpy as jnp
from jax import lax
from jax.experimental import pallas as pl
from jax.experimental.pallas import tpu as pltpu
from jax.experimental.pallas import tpu_sc as plsc

D_MODEL = 768
E = 64
CHUNK = 2048
_SIGN = -2147483648


def _key_of(a):
    ki = lax.bitcast_convert_type(a, jnp.int32)
    ku = lax.bitcast_convert_type(a, jnp.uint32)
    kun = jnp.where(ki < 0, ~ku, ku | jnp.uint32(0x80000000))
    return lax.bitcast_convert_type(kun, jnp.int32) ^ jnp.int32(_SIGN)


def _unkey(k):
    ku = lax.bitcast_convert_type(k ^ jnp.int32(_SIGN), jnp.uint32)
    top = (ku & jnp.uint32(0x80000000)) != 0
    u = jnp.where(top, ku ^ jnp.uint32(0x80000000), ~ku)
    return lax.bitcast_convert_type(u, jnp.float32)


def _unkey_sc(k):
    pat = k ^ jnp.int32(_SIGN)
    fb = jnp.where(k >= 0, pat & jnp.int32(0x7FFFFFFF), ~pat)
    return lax.bitcast_convert_type(fb, jnp.float32)


def _matmul_body(x0_ref, x1_ref, w_ref, keyf_ref, keye_ref):
    dn = (((1,), (1,)), ((), ()))
    a0 = lax.dot_general(x0_ref[...], w_ref[...], dimension_numbers=dn,
                         preferred_element_type=jnp.float32)
    a1 = lax.dot_general(x1_ref[...], w_ref[...], dimension_numbers=dn,
                         preferred_element_type=jnp.float32)
    k0 = _key_of(a0)
    k1 = _key_of(a1)
    keyf_ref[...] = jnp.concatenate([k0, k1], axis=1)
    keye_ref[0] = k0.T
    keye_ref[1] = k1.T


def _affinity_key(tokens, w_sel, *, interpret=False):
    n, d = tokens.shape
    half = n // 2
    T = min(2048, half)
    nb = half // T
    return pl.pallas_call(
        _matmul_body,
        grid=(nb,),
        in_specs=[
            pl.BlockSpec((T, d), lambda i: (i, 0)),
            pl.BlockSpec((T, d), lambda i: (i + nb, 0)),
            pl.BlockSpec((E, d), lambda i: (0, 0)),
        ],
        out_specs=[
            pl.BlockSpec((T, 2 * E), lambda i: (i, 0)),
            pl.BlockSpec((2, E, T), lambda i: (0, 0, i)),
        ],
        out_shape=[
            jax.ShapeDtypeStruct((half, 2 * E), jnp.int32),
            jax.ShapeDtypeStruct((2, E, half), jnp.int32),
        ],
        interpret=interpret,
    )(tokens, tokens, w_sel)


def _splat_sum(v):
    return plsc.cummax(lax.rev(plsc.cumsum(v), (0,)))


def _splat_max(v):
    return plsc.cummax(lax.rev(plsc.cummax(v), (0,)))


def _sc_select(keye, cap):
    _, e, half = keye.shape
    n = 2 * half
    nv = n // 16
    U = 8
    mesh = plsc.VectorSubcoreMesh(core_axis_name="c", subcore_axis_name="s")

    @functools.partial(
        pl.kernel,
        out_type=[
            jax.ShapeDtypeStruct((32, 16), jnp.int32),
            jax.ShapeDtypeStruct((32, 16), jnp.float32),
        ],
        scratch_types=[
            pltpu.VMEM((n,), jnp.int32),
            pltpu.VMEM((n,), jnp.int32),
            pltpu.VMEM((16 * 257,), jnp.int32),
            pltpu.VMEM((256,), jnp.int32),
            pltpu.VMEM((16,), jnp.int32),
            pltpu.VMEM((16,), jnp.float32),
            pltpu.SemaphoreType.DMA,
            pltpu.SemaphoreType.DMA,
        ],
        compiler_params=pltpu.CompilerParams(needs_layout_passes=False),
        mesh=mesh,
    )
    def body(keye_hbm, out_i, out_f, row0, row1, hist, tot, rowi_v, rowf_v,
             sem0, sem1):
        wid = lax.axis_index("s") * 2 + lax.axis_index("c")
        e0 = 2 * wid
        lanes = lax.iota(jnp.int32, 16)
        ones = jnp.ones((16,), jnp.int32)

        c00 = pltpu.async_copy(keye_hbm.at[0, e0], row0.at[pl.ds(0, half)], sem0)
        c01 = pltpu.async_copy(keye_hbm.at[1, e0], row0.at[pl.ds(half, half)], sem0)
        c10 = pltpu.async_copy(keye_hbm.at[0, e0 + 1], row1.at[pl.ds(0, half)], sem1)
        c11 = pltpu.async_copy(keye_hbm.at[1, e0 + 1], row1.at[pl.ds(half, half)], sem1)

        lane_off = lanes * 257

        def zero_hist():
            def zcol(i, carry):
                for l in range(16):
                    hist[pl.ds(l * 257 + i * 16, 16)] = jnp.zeros((16,), jnp.int32)
                return carry
            lax.fori_loop(0, 16, zcol, 0)

        def digit_find(needv):
            scs = jnp.zeros((16,), jnp.int32)

            def totals(i, scs):
                acc = hist[pl.ds(i * 16, 16)]
                for l in range(1, 16):
                    acc = acc + hist[pl.ds(l * 257 + i * 16, 16)]
                tot[pl.ds(i * 16, 16)] = acc
                return scs + acc

            scs = lax.fori_loop(0, 16, totals, scs)
            rv = lax.rev(scs, (0,))
            ic = plsc.cumsum(rv)
            ex = ic - rv
            cond = jnp.logical_and(ex < needv, ic >= needv)
            c_star = _splat_sum(jnp.where(cond, 15 - lanes, 0))
            ssuf = _splat_sum(jnp.where(cond, ex, 0))
            t16 = plsc.load_gather(tot, [lanes * 16 + c_star])
            tr = lax.rev(t16, (0,))
            tc = plsc.cumsum(tr)
            te = tc - tr
            before = ssuf + te
            cond2 = jnp.logical_and(before < needv, before + tr >= needv)
            digit = _splat_sum(jnp.where(cond2, c_star * 16 + 15 - lanes, 0))
            above = _splat_sum(jnp.where(cond2, before, 0))
            return digit, above

        def swap_digit(d):
            return ((d & jnp.int32(15)) << 4) | lax.shift_right_logical(d, 4)

        def process(row):
            zero_hist()

            def scan0(i, m):
                for u in range(U):
                    k = row[pl.ds(i * (16 * U) + u * 16, 16)]
                    pat = k ^ jnp.int32(_SIGN)
                    d = lax.shift_right_logical(pat, 24)
                    plsc.addupdate_scatter(hist, [lane_off + swap_digit(d)], ones)
                    m = jnp.maximum(m, k)
                return m

            m = lax.fori_loop(0, nv // U, scan0,
                              jnp.full((16,), _SIGN, jnp.int32))
            mxk = _splat_max(m)

            needv = jnp.full((16,), cap, jnp.int32)
            digit, above = digit_find(needv)
            pfx = digit
            needv = needv - above

            for p in range(1, 4):
                shift = 24 - 8 * p
                zero_hist()

                def scanp(i, carry, shift=shift, pfx=pfx, row=row):
                    for u in range(U):
                        k = row[pl.ds(i * (16 * U) + u * 16, 16)]
                        pat = k ^ jnp.int32(_SIGN)
                        d = lax.shift_right_logical(pat, shift) & jnp.int32(255)
                        msk = lax.shift_right_logical(pat, shift + 8) == pfx
                        plsc.addupdate_scatter(hist, [lane_off + swap_digit(d)],
                                               ones, mask=msk)
                    return carry

                lax.fori_loop(0, nv // U, scanp, 0)
                digit, above = digit_find(needv)
                pfx = (pfx << 8) | digit
                needv = needv - above

            thr_s = pfx ^ jnp.int32(_SIGN)

            mxf = _unkey_sc(mxk)

            def scane(i, acc):
                for u in range(U):
                    k = row[pl.ds(i * (16 * U) + u * 16, 16)]
                    sel = k >= thr_s
                    a = _unkey_sc(k)
                    acc = acc + jnp.where(sel, jnp.exp(a - mxf), 0.0)
                return acc

            acc = lax.fori_loop(0, nv // U, scane,
                                jnp.zeros((16,), jnp.float32))
            return thr_s, mxk, _splat_sum(acc)

        c00.wait()
        c01.wait()
        thr0, mxk0, s0 = process(row0)
        c10.wait()
        c11.wait()
        thr1, mxk1, s1 = process(row1)

        rowi_v[...] = jnp.where(
            lanes == 0, thr0,
            jnp.where(lanes == 1, thr1,
                      jnp.where(lanes == 2, mxk0,
                                jnp.where(lanes == 3, mxk1, 0))))
        rowf_v[...] = jnp.where(lanes == 0, s0,
                                jnp.where(lanes == 1, s1, 0.0))
        pltpu.sync_copy(rowi_v, out_i.at[wid])
        pltpu.sync_copy(rowf_v, out_f.at[wid])

    return body(keye)


def _finalize_body(key_ref, thr_ref, mxk_ref, s_ref, w_out_ref, s_out_ref):
    n2 = key_ref.shape[0]
    chunk = min(CHUNK, n2)
    nchunks = n2 // chunk
    L = 2 * E

    thr = jnp.concatenate([thr_ref[...], thr_ref[...]], axis=1)
    mx = _unkey(jnp.concatenate([mxk_ref[...], mxk_ref[...]], axis=1))
    s = jnp.concatenate([s_ref[...], s_ref[...]], axis=1)

    lane = lax.broadcasted_iota(jnp.int32, (1, L), 1)
    half0 = lane < E

    def out_step(c, carry):
        rows = pl.ds(c * chunk, chunk)
        k = key_ref[rows, :]
        sel = k >= thr
        self_f = sel.astype(jnp.float32)
        p = jnp.where(sel, jnp.exp(_unkey(k) - mx), 0.0)
        cnt0 = jnp.sum(jnp.where(half0, self_f, 0.0), axis=1, keepdims=True)
        cnt1 = jnp.sum(jnp.where(half0, 0.0, self_f), axis=1, keepdims=True)
        div = jnp.maximum(jnp.where(half0, cnt0, cnt1), 1.0)
        w = p / (s * div)
        r0 = pl.ds(c * chunk, chunk)
        r1 = pl.ds(n2 + c * chunk, chunk)
        w_out_ref[r0, :] = w[:, :E]
        w_out_ref[r1, :] = w[:, E:]
        s_out_ref[r0, :] = self_f[:, :E]
        s_out_ref[r1, :] = self_f[:, E:]
        return carry

    lax.fori_loop(0, nchunks, out_step, 0)


def _finalize(key_folded, thr, mxk, s, *, interpret=False):
    n2, l = key_folded.shape
    n = n2 * 2
    return pl.pallas_call(
        _finalize_body,
        out_shape=(
            jax.ShapeDtypeStruct((n, E), jnp.float32),
            jax.ShapeDtypeStruct((n, E), jnp.float32),
        ),
        interpret=interpret,
    )(key_folded, thr, mxk, s)


def kernel(hidden_states, W_sel):
    batch, seq, d = hidden_states.shape
    num_tokens = batch * seq
    cap = int(num_tokens / E)
    c = min(cap, num_tokens)
    tokens = hidden_states.reshape(num_tokens, d)
    keyf, keye = _affinity_key(tokens, W_sel)
    out_i, out_f = _sc_select(keye, c)
    thr = out_i[:, 0:2].reshape(1, E)
    mxk = out_i[:, 2:4].reshape(1, E)
    s = out_f[:, 0:2].reshape(1, E)
    weights, assign = _finalize(keyf, thr, mxk, s)
    return (weights, assign, cap)


# speedup vs baseline: 1.8450x; 1.7076x over previous
import functools

import jax
import jax.numpy as jnp
from jax import lax
from jax.experimental import pallas as pl
from jax.experimental.pallas import tpu as pltpu
from jax.experimental.pallas import tpu_sc as plsc

D_MODEL = 768
E = 64
CHUNK = 2048
_SIGN = -2147483648


def _key_of(a):
    ki = lax.bitcast_convert_type(a, jnp.int32)
    ku = lax.bitcast_convert_type(a, jnp.uint32)
    kun = jnp.where(ki < 0, ~ku, ku | jnp.uint32(0x80000000))
    return lax.bitcast_convert_type(kun, jnp.int32) ^ jnp.int32(_SIGN)


def _unkey(k):
    ku = lax.bitcast_convert_type(k ^ jnp.int32(_SIGN), jnp.uint32)
    top = (ku & jnp.uint32(0x80000000)) != 0
    u = jnp.where(top, ku ^ jnp.uint32(0x80000000), ~ku)
    return lax.bitcast_convert_type(u, jnp.float32)


def _unkey_sc(k):
    pat = k ^ jnp.int32(_SIGN)
    fb = jnp.where(k >= 0, pat & jnp.int32(0x7FFFFFFF), ~pat)
    return lax.bitcast_convert_type(fb, jnp.float32)


def _matmul_body(x0_ref, x1_ref, w_ref, keyf_ref, keye_ref):
    dn = (((1,), (1,)), ((), ()))
    a0 = lax.dot_general(x0_ref[...], w_ref[...], dimension_numbers=dn,
                         preferred_element_type=jnp.float32)
    a1 = lax.dot_general(x1_ref[...], w_ref[...], dimension_numbers=dn,
                         preferred_element_type=jnp.float32)
    k0 = _key_of(a0)
    k1 = _key_of(a1)
    keyf_ref[...] = jnp.concatenate([k0, k1], axis=1)
    keye_ref[0] = k0.T
    keye_ref[1] = k1.T


def _affinity_key(tokens, w_sel, *, interpret=False):
    n, d = tokens.shape
    half = n // 2
    T = min(2048, half)
    nb = half // T
    return pl.pallas_call(
        _matmul_body,
        grid=(nb,),
        in_specs=[
            pl.BlockSpec((T, d), lambda i: (i, 0)),
            pl.BlockSpec((T, d), lambda i: (i + nb, 0)),
            pl.BlockSpec((E, d), lambda i: (0, 0)),
        ],
        out_specs=[
            pl.BlockSpec((T, 2 * E), lambda i: (i, 0)),
            pl.BlockSpec((2, E, T), lambda i: (0, 0, i)),
        ],
        out_shape=[
            jax.ShapeDtypeStruct((half, 2 * E), jnp.int32),
            jax.ShapeDtypeStruct((2, E, half), jnp.int32),
        ],
        interpret=interpret,
    )(tokens, tokens, w_sel)


def _splat_sum(v):
    return plsc.cummax(lax.rev(plsc.cumsum(v), (0,)))


def _splat_max(v):
    return plsc.cummax(lax.rev(plsc.cummax(v), (0,)))


def _sc_select(keye, cap):
    _, e, half = keye.shape
    n = 2 * half
    nv = n // 16
    U = 8
    mesh = plsc.VectorSubcoreMesh(core_axis_name="c", subcore_axis_name="s")

    @functools.partial(
        pl.kernel,
        out_type=[
            jax.ShapeDtypeStruct((32, 16), jnp.int32),
            jax.ShapeDtypeStruct((32, 16), jnp.float32),
        ],
        scratch_types=[
            pltpu.VMEM((n,), jnp.int32),
            pltpu.VMEM((n,), jnp.int32),
            pltpu.VMEM((16 * 257,), jnp.int32),
            pltpu.VMEM((256,), jnp.int32),
            pltpu.VMEM((16,), jnp.int32),
            pltpu.VMEM((16,), jnp.float32),
            pltpu.SemaphoreType.DMA,
            pltpu.SemaphoreType.DMA,
        ],
        compiler_params=pltpu.CompilerParams(needs_layout_passes=False),
        mesh=mesh,
    )
    def body(keye_hbm, out_i, out_f, row0, row1, hist, tot, rowi_v, rowf_v,
             sem0, sem1):
        wid = lax.axis_index("s") * 2 + lax.axis_index("c")
        e0 = 2 * wid
        lanes = lax.iota(jnp.int32, 16)
        ones = jnp.ones((16,), jnp.int32)

        c00 = pltpu.async_copy(keye_hbm.at[0, e0], row0.at[pl.ds(0, half)], sem0)
        c01 = pltpu.async_copy(keye_hbm.at[1, e0], row0.at[pl.ds(half, half)], sem0)
        c10 = pltpu.async_copy(keye_hbm.at[0, e0 + 1], row1.at[pl.ds(0, half)], sem1)
        c11 = pltpu.async_copy(keye_hbm.at[1, e0 + 1], row1.at[pl.ds(half, half)], sem1)

        lane_off = lanes * 257

        def zero_hist():
            def zcol(i, carry):
                for l in range(16):
                    hist[pl.ds(l * 257 + i * 16, 16)] = jnp.zeros((16,), jnp.int32)
                return carry
            lax.fori_loop(0, 16, zcol, 0)

        def digit_find(needv):
            scs = jnp.zeros((16,), jnp.int32)

            def totals(i, scs):
                acc = hist[pl.ds(i * 16, 16)]
                for l in range(1, 16):
                    acc = acc + hist[pl.ds(l * 257 + i * 16, 16)]
                tot[pl.ds(i * 16, 16)] = acc
                return scs + acc

            scs = lax.fori_loop(0, 16, totals, scs)
            rv = lax.rev(scs, (0,))
            ic = plsc.cumsum(rv)
            ex = ic - rv
            cond = jnp.logical_and(ex < needv, ic >= needv)
            c_star = _splat_sum(jnp.where(cond, 15 - lanes, 0))
            ssuf = _splat_sum(jnp.where(cond, ex, 0))
            t16 = plsc.load_gather(tot, [lanes * 16 + c_star])
            tr = lax.rev(t16, (0,))
            tc = plsc.cumsum(tr)
            te = tc - tr
            before = ssuf + te
            cond2 = jnp.logical_and(before < needv, before + tr >= needv)
            digit = _splat_sum(jnp.where(cond2, c_star * 16 + 15 - lanes, 0))
            above = _splat_sum(jnp.where(cond2, before, 0))
            return digit, above

        def swap_digit(d):
            return ((d & jnp.int32(15)) << 4) | lax.shift_right_logical(d, 4)

        def process(row):
            zero_hist()

            @plsc.parallel_loop(0, nv, unroll=U,
                                carry=jnp.full((16,), _SIGN, jnp.int32))
            def scan0(i, m):
                k = row[pl.ds(i * 16, 16)]
                pat = k ^ jnp.int32(_SIGN)
                d = lax.shift_right_logical(pat, 24)
                plsc.addupdate_scatter(hist, [lane_off + swap_digit(d)], ones)
                return jnp.maximum(m, k)

            mxk = _splat_max(scan0)

            needv = jnp.full((16,), cap, jnp.int32)
            digit, above = digit_find(needv)
            pfx = digit
            needv = needv - above

            for p in range(1, 4):
                shift = 24 - 8 * p
                zero_hist()

                @plsc.parallel_loop(0, nv, unroll=U)
                def scanp(i, shift=shift, pfx=pfx, row=row):
                    k = row[pl.ds(i * 16, 16)]
                    pat = k ^ jnp.int32(_SIGN)
                    d = lax.shift_right_logical(pat, shift) & jnp.int32(255)
                    msk = lax.shift_right_logical(pat, shift + 8) == pfx
                    plsc.addupdate_scatter(hist, [lane_off + swap_digit(d)],
                                           ones, mask=msk)
                digit, above = digit_find(needv)
                pfx = (pfx << 8) | digit
                needv = needv - above

            thr_s = pfx ^ jnp.int32(_SIGN)

            mxf = _unkey_sc(mxk)

            @plsc.parallel_loop(0, nv, unroll=U,
                                carry=jnp.zeros((16,), jnp.float32))
            def scane(i, acc):
                k = row[pl.ds(i * 16, 16)]
                sel = k >= thr_s
                a = _unkey_sc(k)
                return acc + jnp.where(sel, jnp.exp(a - mxf), 0.0)

            return thr_s, mxk, _splat_sum(scane)

        c00.wait()
        c01.wait()
        thr0, mxk0, s0 = process(row0)
        c10.wait()
        c11.wait()
        thr1, mxk1, s1 = process(row1)

        rowi_v[...] = jnp.where(
            lanes == 0, thr0,
            jnp.where(lanes == 1, thr1,
                      jnp.where(lanes == 2, mxk0,
                                jnp.where(lanes == 3, mxk1, 0))))
        rowf_v[...] = jnp.where(lanes == 0, s0,
                                jnp.where(lanes == 1, s1, 0.0))
        pltpu.sync_copy(rowi_v, out_i.at[wid])
        pltpu.sync_copy(rowf_v, out_f.at[wid])

    return body(keye)


def _finalize_body(key_ref, thr_ref, mxk_ref, s_ref, w_out_ref, s_out_ref):
    n2 = key_ref.shape[0]
    chunk = min(CHUNK, n2)
    nchunks = n2 // chunk
    L = 2 * E

    thr = jnp.concatenate([thr_ref[...], thr_ref[...]], axis=1)
    mx = _unkey(jnp.concatenate([mxk_ref[...], mxk_ref[...]], axis=1))
    s = jnp.concatenate([s_ref[...], s_ref[...]], axis=1)

    lane = lax.broadcasted_iota(jnp.int32, (1, L), 1)
    half0 = lane < E

    def out_step(c, carry):
        rows = pl.ds(c * chunk, chunk)
        k = key_ref[rows, :]
        sel = k >= thr
        self_f = sel.astype(jnp.float32)
        p = jnp.where(sel, jnp.exp(_unkey(k) - mx), 0.0)
        cnt0 = jnp.sum(jnp.where(half0, self_f, 0.0), axis=1, keepdims=True)
        cnt1 = jnp.sum(jnp.where(half0, 0.0, self_f), axis=1, keepdims=True)
        div = jnp.maximum(jnp.where(half0, cnt0, cnt1), 1.0)
        w = p / (s * div)
        r0 = pl.ds(c * chunk, chunk)
        r1 = pl.ds(n2 + c * chunk, chunk)
        w_out_ref[r0, :] = w[:, :E]
        w_out_ref[r1, :] = w[:, E:]
        s_out_ref[r0, :] = self_f[:, :E]
        s_out_ref[r1, :] = self_f[:, E:]
        return carry

    lax.fori_loop(0, nchunks, out_step, 0)


def _finalize(key_folded, thr, mxk, s, *, interpret=False):
    n2, l = key_folded.shape
    n = n2 * 2
    return pl.pallas_call(
        _finalize_body,
        out_shape=(
            jax.ShapeDtypeStruct((n, E), jnp.float32),
            jax.ShapeDtypeStruct((n, E), jnp.float32),
        ),
        interpret=interpret,
    )(key_folded, thr, mxk, s)


def kernel(hidden_states, W_sel):
    batch, seq, d = hidden_states.shape
    num_tokens = batch * seq
    cap = int(num_tokens / E)
    c = min(cap, num_tokens)
    tokens = hidden_states.reshape(num_tokens, d)
    keyf, keye = _affinity_key(tokens, W_sel)
    out_i, out_f = _sc_select(keye, c)
    thr = out_i[:, 0:2].reshape(1, E)
    mxk = out_i[:, 2:4].reshape(1, E)
    s = out_f[:, 0:2].reshape(1, E)
    weights, assign = _finalize(keyf, thr, mxk, s)
    return (weights, assign, cap)
